# baseline (speedup 1.0000x reference)
import jax
import jax.numpy as jnp
from jax.experimental import pallas as pl
from jax.experimental.pallas import tpu as pltpu

B, D_IN, H, N_LABELS = 4096, 5000, 1000, 8192
BM1 = 256
BM2 = 512
BN2 = 1024


def _l1_kernel(x_ref, w1_ref, b1_ref, y_ref):
    x = x_ref[...].astype(jnp.bfloat16)
    w = w1_ref[...].astype(jnp.bfloat16)
    acc = jax.lax.dot_general(x, w, (((1,), (1,)), ((), ())),
                              preferred_element_type=jnp.float32)
    y_ref[...] = acc + b1_ref[...]


def _loss_kernel(y_ref, w2_ref, b2_ref, t_ref, out_ref):
    m = pl.program_id(0)
    n = pl.program_id(1)
    hid = jnp.maximum(y_ref[...], 0.0).astype(jnp.bfloat16)
    w = w2_ref[...].astype(jnp.bfloat16)
    z = jax.lax.dot_general(hid, w, (((1,), (1,)), ((), ())),
                            preferred_element_type=jnp.float32)
    z = z + b2_ref[...]
    t = t_ref[...]
    e = jnp.maximum(z, 0.0) - z * t + jnp.log1p(jnp.exp(-jnp.abs(z)))
    s = jnp.sum(e)

    @pl.when((m == 0) & (n == 0))
    def _():
        out_ref[0, 0] = 0.0

    out_ref[0, 0] += s


def kernel(inputs, labels, W1, b1, W2, b2):
    x = inputs.astype(jnp.float32)
    b1r = b1.reshape(1, H)
    b2r = b2.reshape(1, N_LABELS)

    y = pl.pallas_call(
        _l1_kernel,
        grid=(B // BM1,),
        in_specs=[
            pl.BlockSpec((BM1, D_IN), lambda i: (i, 0)),
            pl.BlockSpec((H, D_IN), lambda i: (0, 0)),
            pl.BlockSpec((1, H), lambda i: (0, 0)),
        ],
        out_specs=pl.BlockSpec((BM1, H), lambda i: (i, 0)),
        out_shape=jax.ShapeDtypeStruct((B, H), jnp.float32),
    )(x, W1, b1r)

    loss_sum = pl.pallas_call(
        _loss_kernel,
        grid=(B // BM2, N_LABELS // BN2),
        in_specs=[
            pl.BlockSpec((BM2, H), lambda m, n: (m, 0)),
            pl.BlockSpec((BN2, H), lambda m, n: (n, 0)),
            pl.BlockSpec((1, BN2), lambda m, n: (0, n)),
            pl.BlockSpec((BM2, BN2), lambda m, n: (m, n)),
        ],
        out_specs=pl.BlockSpec(memory_space=pltpu.SMEM),
        out_shape=jax.ShapeDtypeStruct((1, 1), jnp.float32),
    )(y, W2, b2r, labels)

    loss = loss_sum[0, 0] / (B * N_LABELS)
    return (y, loss)
